# baseline (speedup 1.0000x reference)
import functools

import jax
import jax.numpy as jnp
from jax import lax
from jax.experimental import pallas as pl
from jax.experimental.pallas import tpu as pltpu
from jax.experimental.pallas import tpu_sc as plsc

N = 10000
E = 320000
D = 128
H = 64
G = 16
OUT = 2

NC = 2
NS = 16
LN = 16
NW = NC * NS

CH = 128
NCHUNK = 80
EP = NW * NCHUNK * CH
C2 = 160
NP = 10240
RPT = NP // NS

@functools.cache
def _sc_mesh():
    return plsc.VectorSubcoreMesh(core_axis_name="c", subcore_axis_name="s",
                                  num_cores=NC, num_subcores=NS)


def _zero_rows(rows):
    z16 = jnp.zeros((LN,), jnp.float32)

    def body(i, carry):
        r = i // (H // LN)
        q = i % (H // LN)
        rows[r, pl.ds(q * LN, LN)] = z16
        return carry

    lax.fori_loop(0, CH * (H // LN), body, 0, unroll=8)


def _deg_body(idx_hbm, w_hbm, out_hbm, acc, idx_v, w_v, zrow):
    cid = lax.axis_index("c")
    sid = lax.axis_index("s")

    z16 = jnp.zeros((LN,), jnp.float32)

    def zb(i, c):
        zrow[pl.ds(i * LN, LN)] = z16
        return c

    lax.fori_loop(0, RPT // LN, zb, 0, unroll=8)
    pltpu.sync_copy(zrow, acc.at[pl.ds(sid * RPT, RPT)])
    plsc.subcore_barrier()

    pltpu.sync_copy(idx_hbm.at[cid, sid], idx_v)
    pltpu.sync_copy(w_hbm.at[cid, sid], w_v)

    def body(j, c):
        pltpu.sync_copy(w_v.at[j], acc.at[idx_v.at[j]], add=True)
        return c

    lax.fori_loop(0, C2, body, 0)
    plsc.subcore_barrier()
    pltpu.sync_copy(acc.at[pl.ds(sid * RPT, RPT)],
                    out_hbm.at[cid, pl.ds(sid * RPT, RPT)])


@functools.cache
def _deg_kernel():
    return pl.kernel(
        _deg_body,
        out_type=jax.ShapeDtypeStruct((NC, NP), jnp.float32),
        mesh=_sc_mesh(),
        compiler_params=pltpu.CompilerParams(use_tc_tiling_on_sc=False),
        scratch_types=[
            pltpu.VMEM_SHARED((NP,), jnp.float32),
            pltpu.VMEM((C2, CH), jnp.int32),
            pltpu.VMEM((C2, CH), jnp.float32),
            pltpu.VMEM((RPT,), jnp.float32),
        ],
    )


def _scale_rows(rows, rowsf, w_v, j):
    for gi in range(CH // LN):
        wv = w_v[j, pl.ds(gi * LN, LN)]
        for ee in range(LN):
            wb = jnp.full((LN,), wv[ee], jnp.float32)
            e = gi * LN + ee
            for q in range(H // LN):
                rowsf[e, pl.ds(q * LN, LN)] = rows[e, pl.ds(q * LN, LN)] * wb


def _spmm_body(g_hbm, sidx_hbm, didx_hbm, w_hbm, out_hbm,
               acc, sidx_v, didx_v, w_v, rows0, rows1, rowsf, zbuf,
               sem0, sem1):
    cid = lax.axis_index("c")
    sid = lax.axis_index("s")
    wid = sid * NC + cid

    pltpu.sync_copy(sidx_hbm.at[wid], sidx_v)
    pltpu.sync_copy(didx_hbm.at[wid], didx_v)
    pltpu.sync_copy(w_hbm.at[wid], w_v)

    pltpu.async_copy(g_hbm.at[sidx_v.at[0]], rows0, sem0)

    _zero_rows(zbuf)
    for k in range(RPT // CH):
        pltpu.sync_copy(zbuf, acc.at[pl.ds(sid * RPT + k * CH, CH)])
    plsc.subcore_barrier()

    def body(jj, c):
        j0 = 2 * jj
        pltpu.async_copy(g_hbm.at[sidx_v.at[j0 + 1]], rows1, sem1)
        pltpu.make_async_copy(g_hbm.at[sidx_v.at[j0]], rows0, sem0).wait()
        _scale_rows(rows0, rowsf, w_v, j0)
        pltpu.sync_copy(rowsf, acc.at[didx_v.at[j0]], add=True)

        @pl.when(j0 + 2 < NCHUNK)
        def _():
            pltpu.async_copy(g_hbm.at[sidx_v.at[j0 + 2]], rows0, sem0)

        pltpu.make_async_copy(g_hbm.at[sidx_v.at[j0 + 1]], rows1, sem1).wait()
        _scale_rows(rows1, rowsf, w_v, j0 + 1)
        pltpu.sync_copy(rowsf, acc.at[didx_v.at[j0 + 1]], add=True)
        return c

    lax.fori_loop(0, NCHUNK // 2, body, 0)

    plsc.subcore_barrier()
    pltpu.sync_copy(acc.at[pl.ds(sid * RPT, RPT)],
                    out_hbm.at[cid, pl.ds(sid * RPT, RPT)])


@functools.cache
def _spmm_kernel():
    return pl.kernel(
        _spmm_body,
        out_type=jax.ShapeDtypeStruct((NC, NP, H), jnp.float32),
        mesh=_sc_mesh(),
        compiler_params=pltpu.CompilerParams(use_tc_tiling_on_sc=False),
        scratch_types=[
            pltpu.VMEM_SHARED((NP, H), jnp.float32),
            pltpu.VMEM((NCHUNK, CH), jnp.int32),
            pltpu.VMEM((NCHUNK, CH), jnp.int32),
            pltpu.VMEM((NCHUNK, CH), jnp.float32),
            pltpu.VMEM((CH, H), jnp.float32),
            pltpu.VMEM((CH, H), jnp.float32),
            pltpu.VMEM((CH, H), jnp.float32),
            pltpu.VMEM((CH, H), jnp.float32),
            pltpu.SemaphoreType.DMA,
            pltpu.SemaphoreType.DMA,
        ],
    )


def _spmm2_body(g2_hbm, sidx_hbm, didx_hbm, w_hbm, out_hbm,
                acc, sidx_v, didx_v, w_v, rows0, rows1, rowsf,
                sem0, sem1):
    cid = lax.axis_index("c")
    sid = lax.axis_index("s")
    g_hbm = g2_hbm.at[cid]

    pltpu.sync_copy(sidx_hbm.at[cid, sid], sidx_v)
    pltpu.sync_copy(didx_hbm.at[cid, sid], didx_v)
    pltpu.sync_copy(w_hbm.at[cid, sid], w_v)

    pltpu.async_copy(g_hbm.at[sidx_v.at[0]], rows0, sem0)

    _zero_rows(rowsf)
    for k in range(RPT // CH):
        pltpu.sync_copy(rowsf, acc.at[pl.ds(sid * RPT + k * CH, CH)])
    plsc.subcore_barrier()

    def body(jj, c):
        j0 = 2 * jj
        pltpu.async_copy(g_hbm.at[sidx_v.at[j0 + 1]], rows1, sem1)
        pltpu.make_async_copy(g_hbm.at[sidx_v.at[j0]], rows0, sem0).wait()
        _scale_rows(rows0, rowsf, w_v, j0)
        pltpu.sync_copy(rowsf, acc.at[didx_v.at[j0]], add=True)

        @pl.when(j0 + 2 < C2)
        def _():
            pltpu.async_copy(g_hbm.at[sidx_v.at[j0 + 2]], rows0, sem0)

        pltpu.make_async_copy(g_hbm.at[sidx_v.at[j0 + 1]], rows1, sem1).wait()
        _scale_rows(rows1, rowsf, w_v, j0 + 1)
        pltpu.sync_copy(rowsf, acc.at[didx_v.at[j0 + 1]], add=True)
        return c

    lax.fori_loop(0, C2 // 2, body, 0)

    plsc.subcore_barrier()
    pltpu.sync_copy(acc.at[pl.ds(sid * RPT, RPT)],
                    out_hbm.at[cid, pl.ds(sid * RPT, RPT)])


@functools.cache
def _spmm2_kernel():
    return pl.kernel(
        _spmm2_body,
        out_type=jax.ShapeDtypeStruct((NC, NP, H), jnp.float32),
        mesh=_sc_mesh(),
        compiler_params=pltpu.CompilerParams(use_tc_tiling_on_sc=False),
        scratch_types=[
            pltpu.VMEM_SHARED((NP, H), jnp.float32),
            pltpu.VMEM((C2, CH), jnp.int32),
            pltpu.VMEM((C2, CH), jnp.int32),
            pltpu.VMEM((C2, CH), jnp.float32),
            pltpu.VMEM((CH, H), jnp.float32),
            pltpu.VMEM((CH, H), jnp.float32),
            pltpu.VMEM((CH, H), jnp.float32),
            pltpu.SemaphoreType.DMA,
            pltpu.SemaphoreType.DMA,
        ],
    )


def _bn_cols(y, g, be):
    m = jnp.mean(y, axis=0, keepdims=True)
    v = jnp.mean(y * y, axis=0, keepdims=True) - m * m
    return (y - m) * lax.rsqrt(v + 1e-5) * g[None, :] + be[None, :]


def _tc_prep_body(x_ref, w_ref, degp_ref, hw_ref, g_ref, dinvf_ref, dinva_ref):
    degf = degp_ref[0, :N] + 1.0
    dega = degp_ref[1, :N] + 1.0
    dinvf = jnp.where(degf > 0, lax.rsqrt(degf), 0.0)[:, None]
    dinva = jnp.where(dega > 0, lax.rsqrt(dega), 0.0)[:, None]
    hw = jnp.dot(x_ref[...], w_ref[...], preferred_element_type=jnp.float32)
    hw_ref[...] = hw
    g_ref[...] = dinvf * hw
    dinvf_ref[...] = dinvf
    dinva_ref[...] = dinva


_tc_prep = pl.pallas_call(
    _tc_prep_body,
    out_shape=[
        jax.ShapeDtypeStruct((N, H), jnp.float32),
        jax.ShapeDtypeStruct((N, H), jnp.float32),
        jax.ShapeDtypeStruct((N, 1), jnp.float32),
        jax.ShapeDtypeStruct((N, 1), jnp.float32),
    ],
)


def _post(aggp_ref, hw_ref, dinv_ref, b_ref, g_ref, be_ref):
    dinv = dinv_ref[...]
    agg = aggp_ref[0, :N, :] + aggp_ref[1, :N, :]
    y = dinv * agg + (dinv * dinv) * hw_ref[...] + b_ref[...][None, :]
    return _bn_cols(y, g_ref[...], be_ref[...])


def _tc_mid0_body(aggp_ref, hw_ref, dinvf_ref, b_ref, g_ref, be_ref, w1_ref,
                  hw1_ref, g1_ref):
    h0 = jax.nn.relu(_post(aggp_ref, hw_ref, dinvf_ref, b_ref, g_ref, be_ref))
    hw1 = jnp.dot(h0, w1_ref[...], preferred_element_type=jnp.float32)
    hw1_ref[...] = hw1
    g1_ref[...] = dinvf_ref[...] * hw1


_tc_mid0 = pl.pallas_call(
    _tc_mid0_body,
    out_shape=[
        jax.ShapeDtypeStruct((N, H), jnp.float32),
        jax.ShapeDtypeStruct((N, H), jnp.float32),
    ],
)


def _tc_mid1_body(aggp_ref, hw_ref, dinvf_ref, dinva_ref, b_ref, g_ref, be_ref,
                  wa_ref, wf_ref, hwa_ref, hwf_ref, g2_ref):
    h1 = _post(aggp_ref, hw_ref, dinvf_ref, b_ref, g_ref, be_ref)
    hwa = jnp.dot(h1, wa_ref[...], preferred_element_type=jnp.float32)
    hwa_ref[...] = hwa
    hwf = jnp.dot(h1, wf_ref[...], preferred_element_type=jnp.float32)
    hwf_ref[...] = hwf
    g2_ref[0] = dinvf_ref[...] * hwf
    g2_ref[1] = dinva_ref[...] * hwa


_tc_mid1 = pl.pallas_call(
    _tc_mid1_body,
    out_shape=[
        jax.ShapeDtypeStruct((N, H), jnp.float32),
        jax.ShapeDtypeStruct((N, H), jnp.float32),
        jax.ShapeDtypeStruct((NC, N, H), jnp.float32),
    ],
)


def _post1(agg_ref, hw_ref, dinv_ref, b_ref, g_ref, be_ref):
    dinv = dinv_ref[...]
    y = (dinv * agg_ref[:N, :] + (dinv * dinv) * hw_ref[...]
         + b_ref[...][None, :])
    return _bn_cols(y, g_ref[...], be_ref[...])


def _mean_pool(h, batch_ref):
    oh = (batch_ref[...] == lax.broadcasted_iota(jnp.int32, (1, G), 1)
          ).astype(jnp.float32)
    cnt = jnp.maximum(jnp.sum(oh, axis=0), 1.0)[:, None]
    dn = (((0,), (0,)), ((), ()))
    return lax.dot_general(oh, h, dn, preferred_element_type=jnp.float32) / cnt


def _tc_pool_body(agg_ref, hw_ref, dinv_ref, b_ref, g_ref, be_ref, batch_ref,
                  pool_ref):
    h = _post1(agg_ref, hw_ref, dinv_ref, b_ref, g_ref, be_ref)
    pool_ref[...] = _mean_pool(h, batch_ref)


_tc_pool = pl.pallas_call(
    _tc_pool_body,
    out_shape=jax.ShapeDtypeStruct((G, H), jnp.float32),
)


def _tc_poolhead_body(agg_ref, hw_ref, dinv_ref, b_ref, g_ref, be_ref,
                      batch_ref, pa_ref, wc1_ref, bc1_ref, wc2_ref, bc2_ref,
                      out_ref):
    hf = _post1(agg_ref, hw_ref, dinv_ref, b_ref, g_ref, be_ref)
    pf = _mean_pool(hf, batch_ref)
    combined = jnp.concatenate([pa_ref[...], pf], axis=1)
    z = jax.nn.relu(jnp.dot(combined, wc1_ref[...],
                            preferred_element_type=jnp.float32)
                    + bc1_ref[...][None, :])
    out_ref[...] = (jnp.dot(z, wc2_ref[...], preferred_element_type=jnp.float32)
                    + bc2_ref[...][None, :])


_tc_poolhead = pl.pallas_call(
    _tc_poolhead_body,
    out_shape=jax.ShapeDtypeStruct((G, OUT), jnp.float32),
)


def _cast_edges(edge_index, edge_attr):
    src = edge_index[0].astype(jnp.int32)
    dst = edge_index[1].astype(jnp.int32)
    w = edge_attr[:, 0].astype(jnp.float32)
    pad = EP - E
    pidx = (jnp.arange(pad, dtype=jnp.int32) * 37) % N
    src = jnp.concatenate([src, pidx])
    dst = jnp.concatenate([dst, pidx])
    w = jnp.concatenate([w, jnp.zeros((pad,), jnp.float32)])
    return src, dst, w


def kernel(x, func_edge_index, func_edge_attr, anat_edge_index, anat_edge_attr,
           batch, W_s0, b_s0, g_s0, be_s0, W_s1, b_s1, g_s1, be_s1,
           W_a, b_a, g_a, be_a, W_f, b_f, g_f, be_f, Wc1, bc1, Wc2, bc2):
    fsrc, fdst, fwt = _cast_edges(func_edge_index, func_edge_attr)
    asrc, adst, awt = _cast_edges(anat_edge_index, anat_edge_attr)
    fs = fsrc.reshape(NW, NCHUNK, CH)
    fd = fdst.reshape(NW, NCHUNK, CH)
    fw = fwt.reshape(NW, NCHUNK, CH)
    src2 = jnp.stack([fsrc, asrc]).reshape(NC, NS, C2, CH)
    dst2 = jnp.stack([fdst, adst]).reshape(NC, NS, C2, CH)
    w2 = jnp.stack([fwt, awt]).reshape(NC, NS, C2, CH)
    batch2d = batch.astype(jnp.int32)[:, None]

    degp = _deg_kernel()(dst2, w2)
    hw0, g0, dinvf, dinva = _tc_prep(x, W_s0, degp)

    spmm = _spmm_kernel()
    aggp0 = spmm(g0, fs, fd, fw)
    hw1, g1 = _tc_mid0(aggp0, hw0, dinvf, b_s0, g_s0, be_s0, W_s1)

    aggp1 = spmm(g1, fs, fd, fw)
    hwa, hwf, g2 = _tc_mid1(aggp1, hw1, dinvf, dinva, b_s1, g_s1, be_s1,
                            W_a, W_f)

    agg2 = _spmm2_kernel()(g2, src2, dst2, w2)

    pa = _tc_pool(agg2[1], hwa, dinva, b_a, g_a, be_a, batch2d)
    return _tc_poolhead(agg2[0], hwf, dinvf, b_f, g_f, be_f, batch2d,
                        pa, Wc1, bc1, Wc2, bc2)


# speedup vs baseline: 1.0208x; 1.0208x over previous
import functools

import jax
import jax.numpy as jnp
from jax import lax
from jax.experimental import pallas as pl
from jax.experimental.pallas import tpu as pltpu
from jax.experimental.pallas import tpu_sc as plsc

N = 10000
E = 320000
D = 128
H = 64
G = 16
OUT = 2

NC = 2
NS = 16
LN = 16
NW = NC * NS

CH = 128
NCHUNK = 80
EP = NW * NCHUNK * CH
C2 = 160
NP = 10240
RPT = NP // NS

@functools.cache
def _sc_mesh():
    return plsc.VectorSubcoreMesh(core_axis_name="c", subcore_axis_name="s",
                                  num_cores=NC, num_subcores=NS)


def _zero_rows(rows):
    z16 = jnp.zeros((LN,), jnp.float32)

    def body(i, carry):
        r = i // (H // LN)
        q = i % (H // LN)
        rows[r, pl.ds(q * LN, LN)] = z16
        return carry

    lax.fori_loop(0, CH * (H // LN), body, 0, unroll=8)


def _deg_body(idx_hbm, w_hbm, out_hbm, acc, idx_v, w_v, zrow):
    cid = lax.axis_index("c")
    sid = lax.axis_index("s")

    z16 = jnp.zeros((LN,), jnp.float32)

    def zb(i, c):
        zrow[pl.ds(i * LN, LN)] = z16
        return c

    lax.fori_loop(0, RPT // LN, zb, 0, unroll=8)
    pltpu.sync_copy(zrow, acc.at[pl.ds(sid * RPT, RPT)])
    plsc.subcore_barrier()

    pltpu.sync_copy(idx_hbm.at[cid, sid], idx_v)
    pltpu.sync_copy(w_hbm.at[cid, sid], w_v)

    def body(j, c):
        pltpu.sync_copy(w_v.at[j], acc.at[idx_v.at[j]], add=True)
        return c

    lax.fori_loop(0, C2, body, 0)
    plsc.subcore_barrier()
    pltpu.sync_copy(acc.at[pl.ds(sid * RPT, RPT)],
                    out_hbm.at[cid, pl.ds(sid * RPT, RPT)])


@functools.cache
def _deg_kernel():
    return pl.kernel(
        _deg_body,
        out_type=jax.ShapeDtypeStruct((NC, NP), jnp.float32),
        mesh=_sc_mesh(),
        compiler_params=pltpu.CompilerParams(use_tc_tiling_on_sc=False),
        scratch_types=[
            pltpu.VMEM_SHARED((NP,), jnp.float32),
            pltpu.VMEM((C2, CH), jnp.int32),
            pltpu.VMEM((C2, CH), jnp.float32),
            pltpu.VMEM((RPT,), jnp.float32),
        ],
    )


def _scale_rows(rows, rowsf, w_v, j):
    for gi in range(CH // LN):
        wv = w_v[j, pl.ds(gi * LN, LN)]
        for ee in range(LN):
            wb = jnp.full((LN,), wv[ee], jnp.float32)
            e = gi * LN + ee
            for q in range(H // LN):
                rowsf[e, pl.ds(q * LN, LN)] = rows[e, pl.ds(q * LN, LN)] * wb


def _spmm_body(g_hbm, sidx_hbm, didx_hbm, w_hbm, out_hbm,
               acc, sidx_v, didx_v, w_v, rows0, rows1, rowsf, zbuf,
               sem0, sem1):
    cid = lax.axis_index("c")
    sid = lax.axis_index("s")
    wid = sid * NC + cid

    pltpu.sync_copy(sidx_hbm.at[wid], sidx_v)
    pltpu.sync_copy(didx_hbm.at[wid], didx_v)
    pltpu.sync_copy(w_hbm.at[wid], w_v)

    pltpu.async_copy(g_hbm.at[sidx_v.at[0]], rows0, sem0)

    _zero_rows(zbuf)
    for k in range(RPT // CH):
        pltpu.sync_copy(zbuf, acc.at[pl.ds(sid * RPT + k * CH, CH)])
    plsc.subcore_barrier()

    def body(jj, c):
        j0 = 2 * jj
        pltpu.async_copy(g_hbm.at[sidx_v.at[j0 + 1]], rows1, sem1)
        pltpu.make_async_copy(g_hbm.at[sidx_v.at[j0]], rows0, sem0).wait()
        _scale_rows(rows0, rowsf, w_v, j0)
        pltpu.sync_copy(rowsf, acc.at[didx_v.at[j0]], add=True)

        @pl.when(j0 + 2 < NCHUNK)
        def _():
            pltpu.async_copy(g_hbm.at[sidx_v.at[j0 + 2]], rows0, sem0)

        pltpu.make_async_copy(g_hbm.at[sidx_v.at[j0 + 1]], rows1, sem1).wait()
        _scale_rows(rows1, rowsf, w_v, j0 + 1)
        pltpu.sync_copy(rowsf, acc.at[didx_v.at[j0 + 1]], add=True)
        return c

    lax.fori_loop(0, NCHUNK // 2, body, 0)

    plsc.subcore_barrier()
    pltpu.sync_copy(acc.at[pl.ds(sid * RPT, RPT)],
                    out_hbm.at[cid, pl.ds(sid * RPT, RPT)])


@functools.cache
def _spmm_kernel():
    return pl.kernel(
        _spmm_body,
        out_type=jax.ShapeDtypeStruct((NC, NP, H), jnp.float32),
        mesh=_sc_mesh(),
        compiler_params=pltpu.CompilerParams(use_tc_tiling_on_sc=False),
        scratch_types=[
            pltpu.VMEM_SHARED((NP, H), jnp.float32),
            pltpu.VMEM((NCHUNK, CH), jnp.int32),
            pltpu.VMEM((NCHUNK, CH), jnp.int32),
            pltpu.VMEM((NCHUNK, CH), jnp.float32),
            pltpu.VMEM((CH, H), jnp.float32),
            pltpu.VMEM((CH, H), jnp.float32),
            pltpu.VMEM((CH, H), jnp.float32),
            pltpu.VMEM((CH, H), jnp.float32),
            pltpu.SemaphoreType.DMA,
            pltpu.SemaphoreType.DMA,
        ],
    )


def _spmm2_body(g2_hbm, sidx_hbm, didx_hbm, w_hbm, out_hbm,
                acc, sidx_v, didx_v, w_v, rows0, rows1, rowsf,
                sem0, sem1):
    cid = lax.axis_index("c")
    sid = lax.axis_index("s")
    g_hbm = g2_hbm.at[cid]

    pltpu.sync_copy(sidx_hbm.at[cid, sid], sidx_v)
    pltpu.sync_copy(didx_hbm.at[cid, sid], didx_v)
    pltpu.sync_copy(w_hbm.at[cid, sid], w_v)

    pltpu.async_copy(g_hbm.at[sidx_v.at[0]], rows0, sem0)

    _zero_rows(rowsf)
    for k in range(RPT // CH):
        pltpu.sync_copy(rowsf, acc.at[pl.ds(sid * RPT + k * CH, CH)])
    plsc.subcore_barrier()

    def body(jj, c):
        j0 = 2 * jj
        pltpu.async_copy(g_hbm.at[sidx_v.at[j0 + 1]], rows1, sem1)
        pltpu.make_async_copy(g_hbm.at[sidx_v.at[j0]], rows0, sem0).wait()
        _scale_rows(rows0, rowsf, w_v, j0)
        pltpu.sync_copy(rowsf, acc.at[didx_v.at[j0]], add=True)

        @pl.when(j0 + 2 < C2)
        def _():
            pltpu.async_copy(g_hbm.at[sidx_v.at[j0 + 2]], rows0, sem0)

        pltpu.make_async_copy(g_hbm.at[sidx_v.at[j0 + 1]], rows1, sem1).wait()
        _scale_rows(rows1, rowsf, w_v, j0 + 1)
        pltpu.sync_copy(rowsf, acc.at[didx_v.at[j0 + 1]], add=True)
        return c

    lax.fori_loop(0, C2 // 2, body, 0)

    plsc.subcore_barrier()
    pltpu.sync_copy(acc.at[pl.ds(sid * RPT, RPT)],
                    out_hbm.at[cid, pl.ds(sid * RPT, RPT)])


@functools.cache
def _spmm2_kernel():
    return pl.kernel(
        _spmm2_body,
        out_type=jax.ShapeDtypeStruct((NC, NP, H), jnp.float32),
        mesh=_sc_mesh(),
        compiler_params=pltpu.CompilerParams(use_tc_tiling_on_sc=False),
        scratch_types=[
            pltpu.VMEM_SHARED((NP, H), jnp.float32),
            pltpu.VMEM((C2, CH), jnp.int32),
            pltpu.VMEM((C2, CH), jnp.int32),
            pltpu.VMEM((C2, CH), jnp.float32),
            pltpu.VMEM((CH, H), jnp.float32),
            pltpu.VMEM((CH, H), jnp.float32),
            pltpu.VMEM((CH, H), jnp.float32),
            pltpu.SemaphoreType.DMA,
            pltpu.SemaphoreType.DMA,
        ],
    )


def _bn_cols(y, g, be):
    m = jnp.mean(y, axis=0, keepdims=True)
    v = jnp.mean(y * y, axis=0, keepdims=True) - m * m
    return (y - m) * lax.rsqrt(v + 1e-5) * g[None, :] + be[None, :]


def _tc_prep_body(x_ref, w_ref, degp_ref, hw_ref, g_ref, dinvf_ref, dinva_ref):
    degf = degp_ref[0, :N] + 1.0
    dega = degp_ref[1, :N] + 1.0
    dinvf = jnp.where(degf > 0, lax.rsqrt(degf), 0.0)[:, None]
    dinva = jnp.where(dega > 0, lax.rsqrt(dega), 0.0)[:, None]
    hw = jnp.dot(x_ref[...], w_ref[...], preferred_element_type=jnp.float32)
    hw_ref[...] = hw
    g_ref[...] = dinvf * hw
    dinvf_ref[...] = dinvf
    dinva_ref[...] = dinva


_tc_prep = pl.pallas_call(
    _tc_prep_body,
    out_shape=[
        jax.ShapeDtypeStruct((N, H), jnp.float32),
        jax.ShapeDtypeStruct((N, H), jnp.float32),
        jax.ShapeDtypeStruct((N, 1), jnp.float32),
        jax.ShapeDtypeStruct((N, 1), jnp.float32),
    ],
)


def _post(aggp_ref, hw_ref, dinv_ref, b_ref, g_ref, be_ref):
    dinv = dinv_ref[...]
    agg = aggp_ref[0, :N, :] + aggp_ref[1, :N, :]
    y = dinv * agg + (dinv * dinv) * hw_ref[...] + b_ref[...][None, :]
    return _bn_cols(y, g_ref[...], be_ref[...])


def _tc_mid0_body(aggp_ref, hw_ref, dinvf_ref, b_ref, g_ref, be_ref, w1_ref,
                  hw1_ref, g1_ref):
    h0 = jax.nn.relu(_post(aggp_ref, hw_ref, dinvf_ref, b_ref, g_ref, be_ref))
    hw1 = jnp.dot(h0, w1_ref[...], preferred_element_type=jnp.float32)
    hw1_ref[...] = hw1
    g1_ref[...] = dinvf_ref[...] * hw1


_tc_mid0 = pl.pallas_call(
    _tc_mid0_body,
    out_shape=[
        jax.ShapeDtypeStruct((N, H), jnp.float32),
        jax.ShapeDtypeStruct((N, H), jnp.float32),
    ],
)


def _tc_mid1_body(aggp_ref, hw_ref, dinvf_ref, dinva_ref, b_ref, g_ref, be_ref,
                  wa_ref, wf_ref, hwa_ref, hwf_ref, g2_ref):
    h1 = _post(aggp_ref, hw_ref, dinvf_ref, b_ref, g_ref, be_ref)
    hwa = jnp.dot(h1, wa_ref[...], preferred_element_type=jnp.float32)
    hwa_ref[...] = hwa
    hwf = jnp.dot(h1, wf_ref[...], preferred_element_type=jnp.float32)
    hwf_ref[...] = hwf
    g2_ref[0] = dinvf_ref[...] * hwf
    g2_ref[1] = dinva_ref[...] * hwa


_tc_mid1 = pl.pallas_call(
    _tc_mid1_body,
    out_shape=[
        jax.ShapeDtypeStruct((N, H), jnp.float32),
        jax.ShapeDtypeStruct((N, H), jnp.float32),
        jax.ShapeDtypeStruct((NC, N, H), jnp.float32),
    ],
)


def _post1(agg_ref, hw_ref, dinv_ref, b_ref, g_ref, be_ref):
    dinv = dinv_ref[...]
    y = (dinv * agg_ref[:N, :] + (dinv * dinv) * hw_ref[...]
         + b_ref[...][None, :])
    return _bn_cols(y, g_ref[...], be_ref[...])


def _mean_pool(h, batch_ref):
    oh = (batch_ref[...] == lax.broadcasted_iota(jnp.int32, (1, G), 1)
          ).astype(jnp.float32)
    cnt = jnp.maximum(jnp.sum(oh, axis=0), 1.0)[:, None]
    dn = (((0,), (0,)), ((), ()))
    return lax.dot_general(oh, h, dn, preferred_element_type=jnp.float32) / cnt


def _tc_final_body(agg2_ref, hwa_ref, hwf_ref, dinva_ref, dinvf_ref,
                   ba_ref, ga_ref, bea_ref, bf_ref, gf_ref, bef_ref,
                   batch_ref, wc1_ref, bc1_ref, wc2_ref, bc2_ref, out_ref):
    ha = _post1(agg2_ref.at[1], hwa_ref, dinva_ref, ba_ref, ga_ref, bea_ref)
    pa = _mean_pool(ha, batch_ref)
    hf = _post1(agg2_ref.at[0], hwf_ref, dinvf_ref, bf_ref, gf_ref, bef_ref)
    pf = _mean_pool(hf, batch_ref)
    combined = jnp.concatenate([pa, pf], axis=1)
    z = jax.nn.relu(jnp.dot(combined, wc1_ref[...],
                            preferred_element_type=jnp.float32)
                    + bc1_ref[...][None, :])
    out_ref[...] = (jnp.dot(z, wc2_ref[...], preferred_element_type=jnp.float32)
                    + bc2_ref[...][None, :])


_tc_final = pl.pallas_call(
    _tc_final_body,
    out_shape=jax.ShapeDtypeStruct((G, OUT), jnp.float32),
)


def _cast_edges(edge_index, edge_attr):
    src = edge_index[0].astype(jnp.int32)
    dst = edge_index[1].astype(jnp.int32)
    w = edge_attr[:, 0].astype(jnp.float32)
    pad = EP - E
    pidx = (jnp.arange(pad, dtype=jnp.int32) * 37) % N
    src = jnp.concatenate([src, pidx])
    dst = jnp.concatenate([dst, pidx])
    w = jnp.concatenate([w, jnp.zeros((pad,), jnp.float32)])
    return src, dst, w


def kernel(x, func_edge_index, func_edge_attr, anat_edge_index, anat_edge_attr,
           batch, W_s0, b_s0, g_s0, be_s0, W_s1, b_s1, g_s1, be_s1,
           W_a, b_a, g_a, be_a, W_f, b_f, g_f, be_f, Wc1, bc1, Wc2, bc2):
    fsrc, fdst, fwt = _cast_edges(func_edge_index, func_edge_attr)
    asrc, adst, awt = _cast_edges(anat_edge_index, anat_edge_attr)
    fs = fsrc.reshape(NW, NCHUNK, CH)
    fd = fdst.reshape(NW, NCHUNK, CH)
    fw = fwt.reshape(NW, NCHUNK, CH)
    src2 = jnp.stack([fsrc, asrc]).reshape(NC, NS, C2, CH)
    dst2 = jnp.stack([fdst, adst]).reshape(NC, NS, C2, CH)
    w2 = jnp.stack([fwt, awt]).reshape(NC, NS, C2, CH)
    batch2d = batch.astype(jnp.int32)[:, None]

    degp = _deg_kernel()(dst2, w2)
    hw0, g0, dinvf, dinva = _tc_prep(x, W_s0, degp)

    spmm = _spmm_kernel()
    aggp0 = spmm(g0, fs, fd, fw)
    hw1, g1 = _tc_mid0(aggp0, hw0, dinvf, b_s0, g_s0, be_s0, W_s1)

    aggp1 = spmm(g1, fs, fd, fw)
    hwa, hwf, g2 = _tc_mid1(aggp1, hw1, dinvf, dinva, b_s1, g_s1, be_s1,
                            W_a, W_f)

    agg2 = _spmm2_kernel()(g2, src2, dst2, w2)

    return _tc_final(agg2, hwa, hwf, dinva, dinvf, b_a, g_a, be_a,
                     b_f, g_f, be_f, batch2d, Wc1, bc1, Wc2, bc2)


# speedup vs baseline: 1.0669x; 1.0451x over previous
import functools

import jax
import jax.numpy as jnp
from jax import lax
from jax.experimental import pallas as pl
from jax.experimental.pallas import tpu as pltpu
from jax.experimental.pallas import tpu_sc as plsc

N = 10000
E = 320000
D = 128
H = 64
G = 16
OUT = 2

NC = 2
NS = 16
LN = 16
NW = NC * NS

CH = 128
NCHUNK = 80
EP = NW * NCHUNK * CH
C2 = 160
NP = 10240
RPT = NP // NS

@functools.cache
def _sc_mesh():
    return plsc.VectorSubcoreMesh(core_axis_name="c", subcore_axis_name="s",
                                  num_cores=NC, num_subcores=NS)


def _zero_rows(rows):
    z16 = jnp.zeros((LN,), jnp.float32)

    def body(i, carry):
        r = i // (H // LN)
        q = i % (H // LN)
        rows[r, pl.ds(q * LN, LN)] = z16
        return carry

    lax.fori_loop(0, CH * (H // LN), body, 0, unroll=8)


def _deg_body(idx_hbm, w_hbm, out_hbm, acc, idx_v, w_v, zrow):
    cid = lax.axis_index("c")
    sid = lax.axis_index("s")

    z16 = jnp.zeros((LN,), jnp.float32)

    def zb(i, c):
        zrow[pl.ds(i * LN, LN)] = z16
        return c

    lax.fori_loop(0, RPT // LN, zb, 0, unroll=8)
    pltpu.sync_copy(zrow, acc.at[pl.ds(sid * RPT, RPT)])
    plsc.subcore_barrier()

    pltpu.sync_copy(idx_hbm.at[cid, sid], idx_v)
    pltpu.sync_copy(w_hbm.at[cid, sid], w_v)

    def body(j, c):
        pltpu.sync_copy(w_v.at[j], acc.at[idx_v.at[j]], add=True)
        return c

    lax.fori_loop(0, C2, body, 0)
    plsc.subcore_barrier()
    pltpu.sync_copy(acc.at[pl.ds(sid * RPT, RPT)],
                    out_hbm.at[cid, pl.ds(sid * RPT, RPT)])


@functools.cache
def _deg_kernel():
    return pl.kernel(
        _deg_body,
        out_type=jax.ShapeDtypeStruct((NC, NP), jnp.float32),
        mesh=_sc_mesh(),
        compiler_params=pltpu.CompilerParams(use_tc_tiling_on_sc=False),
        scratch_types=[
            pltpu.VMEM_SHARED((NP,), jnp.float32),
            pltpu.VMEM((C2, CH), jnp.int32),
            pltpu.VMEM((C2, CH), jnp.float32),
            pltpu.VMEM((RPT,), jnp.float32),
        ],
    )


def _scale_rows(rows, rowsf, w_v, j):
    for gi in range(CH // LN):
        wv = w_v[j, pl.ds(gi * LN, LN)]
        for ee in range(LN):
            wb = jnp.full((LN,), wv[ee], jnp.float32)
            e = gi * LN + ee
            for q in range(H // LN):
                rowsf[e, pl.ds(q * LN, LN)] = rows[e, pl.ds(q * LN, LN)] * wb


def _spmm_body(g_hbm, sidx_hbm, didx_hbm, w_hbm, out_hbm,
               acc, sidx_v, didx_v, w_v, rows0, rows1, rowsf0, rowsf1,
               sem0, sem1, ssem0, ssem1):
    cid = lax.axis_index("c")
    sid = lax.axis_index("s")
    wid = sid * NC + cid

    pltpu.sync_copy(sidx_hbm.at[wid], sidx_v)
    pltpu.sync_copy(didx_hbm.at[wid], didx_v)
    pltpu.sync_copy(w_hbm.at[wid], w_v)

    pltpu.async_copy(g_hbm.at[sidx_v.at[0]], rows0, sem0)

    _zero_rows(rowsf0)
    for k in range(RPT // CH):
        pltpu.sync_copy(rowsf0, acc.at[pl.ds(sid * RPT + k * CH, CH)])
    plsc.subcore_barrier()

    def body(jj, c):
        j0 = 2 * jj
        pltpu.async_copy(g_hbm.at[sidx_v.at[j0 + 1]], rows1, sem1)
        pltpu.make_async_copy(g_hbm.at[sidx_v.at[j0]], rows0, sem0).wait()

        @pl.when(j0 >= 2)
        def _():
            pltpu.make_async_copy(rowsf0, acc.at[didx_v.at[j0 - 2]],
                                  ssem0).wait()

        _scale_rows(rows0, rowsf0, w_v, j0)
        pltpu.async_copy(rowsf0, acc.at[didx_v.at[j0]], ssem0, add=True)

        @pl.when(j0 + 2 < NCHUNK)
        def _():
            pltpu.async_copy(g_hbm.at[sidx_v.at[j0 + 2]], rows0, sem0)

        pltpu.make_async_copy(g_hbm.at[sidx_v.at[j0 + 1]], rows1, sem1).wait()

        @pl.when(j0 >= 2)
        def _():
            pltpu.make_async_copy(rowsf1, acc.at[didx_v.at[j0 - 1]],
                                  ssem1).wait()

        _scale_rows(rows1, rowsf1, w_v, j0 + 1)
        pltpu.async_copy(rowsf1, acc.at[didx_v.at[j0 + 1]], ssem1, add=True)
        return c

    lax.fori_loop(0, NCHUNK // 2, body, 0)

    pltpu.make_async_copy(rowsf0, acc.at[didx_v.at[NCHUNK - 2]], ssem0).wait()
    pltpu.make_async_copy(rowsf1, acc.at[didx_v.at[NCHUNK - 1]], ssem1).wait()

    plsc.subcore_barrier()
    pltpu.sync_copy(acc.at[pl.ds(sid * RPT, RPT)],
                    out_hbm.at[cid, pl.ds(sid * RPT, RPT)])


@functools.cache
def _spmm_kernel():
    return pl.kernel(
        _spmm_body,
        out_type=jax.ShapeDtypeStruct((NC, NP, H), jnp.float32),
        mesh=_sc_mesh(),
        compiler_params=pltpu.CompilerParams(use_tc_tiling_on_sc=False),
        scratch_types=[
            pltpu.VMEM_SHARED((NP, H), jnp.float32),
            pltpu.VMEM((NCHUNK, CH), jnp.int32),
            pltpu.VMEM((NCHUNK, CH), jnp.int32),
            pltpu.VMEM((NCHUNK, CH), jnp.float32),
            pltpu.VMEM((CH, H), jnp.float32),
            pltpu.VMEM((CH, H), jnp.float32),
            pltpu.VMEM((CH, H), jnp.float32),
            pltpu.VMEM((CH, H), jnp.float32),
            pltpu.SemaphoreType.DMA,
            pltpu.SemaphoreType.DMA,
            pltpu.SemaphoreType.DMA,
            pltpu.SemaphoreType.DMA,
        ],
    )


def _spmm2_body(g2_hbm, sidx_hbm, didx_hbm, w_hbm, out_hbm,
                acc, sidx_v, didx_v, w_v, rows0, rows1, rowsf,
                sem0, sem1):
    cid = lax.axis_index("c")
    sid = lax.axis_index("s")
    g_hbm = g2_hbm.at[cid]

    pltpu.sync_copy(sidx_hbm.at[cid, sid], sidx_v)
    pltpu.sync_copy(didx_hbm.at[cid, sid], didx_v)
    pltpu.sync_copy(w_hbm.at[cid, sid], w_v)

    pltpu.async_copy(g_hbm.at[sidx_v.at[0]], rows0, sem0)

    _zero_rows(rowsf)
    for k in range(RPT // CH):
        pltpu.sync_copy(rowsf, acc.at[pl.ds(sid * RPT + k * CH, CH)])
    plsc.subcore_barrier()

    def body(jj, c):
        j0 = 2 * jj
        pltpu.async_copy(g_hbm.at[sidx_v.at[j0 + 1]], rows1, sem1)
        pltpu.make_async_copy(g_hbm.at[sidx_v.at[j0]], rows0, sem0).wait()
        _scale_rows(rows0, rowsf, w_v, j0)
        pltpu.sync_copy(rowsf, acc.at[didx_v.at[j0]], add=True)

        @pl.when(j0 + 2 < C2)
        def _():
            pltpu.async_copy(g_hbm.at[sidx_v.at[j0 + 2]], rows0, sem0)

        pltpu.make_async_copy(g_hbm.at[sidx_v.at[j0 + 1]], rows1, sem1).wait()
        _scale_rows(rows1, rowsf, w_v, j0 + 1)
        pltpu.sync_copy(rowsf, acc.at[didx_v.at[j0 + 1]], add=True)
        return c

    lax.fori_loop(0, C2 // 2, body, 0)

    plsc.subcore_barrier()
    pltpu.sync_copy(acc.at[pl.ds(sid * RPT, RPT)],
                    out_hbm.at[cid, pl.ds(sid * RPT, RPT)])


@functools.cache
def _spmm2_kernel():
    return pl.kernel(
        _spmm2_body,
        out_type=jax.ShapeDtypeStruct((NC, NP, H), jnp.float32),
        mesh=_sc_mesh(),
        compiler_params=pltpu.CompilerParams(use_tc_tiling_on_sc=False),
        scratch_types=[
            pltpu.VMEM_SHARED((NP, H), jnp.float32),
            pltpu.VMEM((C2, CH), jnp.int32),
            pltpu.VMEM((C2, CH), jnp.int32),
            pltpu.VMEM((C2, CH), jnp.float32),
            pltpu.VMEM((CH, H), jnp.float32),
            pltpu.VMEM((CH, H), jnp.float32),
            pltpu.VMEM((CH, H), jnp.float32),
            pltpu.SemaphoreType.DMA,
            pltpu.SemaphoreType.DMA,
        ],
    )


def _bn_cols(y, g, be):
    m = jnp.mean(y, axis=0, keepdims=True)
    v = jnp.mean(y * y, axis=0, keepdims=True) - m * m
    return (y - m) * lax.rsqrt(v + 1e-5) * g[None, :] + be[None, :]


def _tc_prep_body(x_ref, w_ref, degp_ref, hw_ref, g_ref, dinvf_ref, dinva_ref):
    degf = degp_ref[0, :N] + 1.0
    dega = degp_ref[1, :N] + 1.0
    dinvf = jnp.where(degf > 0, lax.rsqrt(degf), 0.0)[:, None]
    dinva = jnp.where(dega > 0, lax.rsqrt(dega), 0.0)[:, None]
    hw = jnp.dot(x_ref[...], w_ref[...], preferred_element_type=jnp.float32)
    hw_ref[...] = hw
    g_ref[...] = dinvf * hw
    dinvf_ref[...] = dinvf
    dinva_ref[...] = dinva


_tc_prep = pl.pallas_call(
    _tc_prep_body,
    out_shape=[
        jax.ShapeDtypeStruct((N, H), jnp.float32),
        jax.ShapeDtypeStruct((N, H), jnp.float32),
        jax.ShapeDtypeStruct((N, 1), jnp.float32),
        jax.ShapeDtypeStruct((N, 1), jnp.float32),
    ],
)


def _post(aggp_ref, hw_ref, dinv_ref, b_ref, g_ref, be_ref):
    dinv = dinv_ref[...]
    agg = aggp_ref[0, :N, :] + aggp_ref[1, :N, :]
    y = dinv * agg + (dinv * dinv) * hw_ref[...] + b_ref[...][None, :]
    return _bn_cols(y, g_ref[...], be_ref[...])


def _tc_mid0_body(aggp_ref, hw_ref, dinvf_ref, b_ref, g_ref, be_ref, w1_ref,
                  hw1_ref, g1_ref):
    h0 = jax.nn.relu(_post(aggp_ref, hw_ref, dinvf_ref, b_ref, g_ref, be_ref))
    hw1 = jnp.dot(h0, w1_ref[...], preferred_element_type=jnp.float32)
    hw1_ref[...] = hw1
    g1_ref[...] = dinvf_ref[...] * hw1


_tc_mid0 = pl.pallas_call(
    _tc_mid0_body,
    out_shape=[
        jax.ShapeDtypeStruct((N, H), jnp.float32),
        jax.ShapeDtypeStruct((N, H), jnp.float32),
    ],
)


def _tc_mid1_body(aggp_ref, hw_ref, dinvf_ref, dinva_ref, b_ref, g_ref, be_ref,
                  wa_ref, wf_ref, hwa_ref, hwf_ref, g2_ref):
    h1 = _post(aggp_ref, hw_ref, dinvf_ref, b_ref, g_ref, be_ref)
    hwa = jnp.dot(h1, wa_ref[...], preferred_element_type=jnp.float32)
    hwa_ref[...] = hwa
    hwf = jnp.dot(h1, wf_ref[...], preferred_element_type=jnp.float32)
    hwf_ref[...] = hwf
    g2_ref[0] = dinvf_ref[...] * hwf
    g2_ref[1] = dinva_ref[...] * hwa


_tc_mid1 = pl.pallas_call(
    _tc_mid1_body,
    out_shape=[
        jax.ShapeDtypeStruct((N, H), jnp.float32),
        jax.ShapeDtypeStruct((N, H), jnp.float32),
        jax.ShapeDtypeStruct((NC, N, H), jnp.float32),
    ],
)


def _post1(agg_ref, hw_ref, dinv_ref, b_ref, g_ref, be_ref):
    dinv = dinv_ref[...]
    y = (dinv * agg_ref[:N, :] + (dinv * dinv) * hw_ref[...]
         + b_ref[...][None, :])
    return _bn_cols(y, g_ref[...], be_ref[...])


def _mean_pool(h, batch_ref):
    oh = (batch_ref[...] == lax.broadcasted_iota(jnp.int32, (1, G), 1)
          ).astype(jnp.float32)
    cnt = jnp.maximum(jnp.sum(oh, axis=0), 1.0)[:, None]
    dn = (((0,), (0,)), ((), ()))
    return lax.dot_general(oh, h, dn, preferred_element_type=jnp.float32) / cnt


def _tc_final_body(agg2_ref, hwa_ref, hwf_ref, dinva_ref, dinvf_ref,
                   ba_ref, ga_ref, bea_ref, bf_ref, gf_ref, bef_ref,
                   batch_ref, wc1_ref, bc1_ref, wc2_ref, bc2_ref, out_ref):
    ha = _post1(agg2_ref.at[1], hwa_ref, dinva_ref, ba_ref, ga_ref, bea_ref)
    pa = _mean_pool(ha, batch_ref)
    hf = _post1(agg2_ref.at[0], hwf_ref, dinvf_ref, bf_ref, gf_ref, bef_ref)
    pf = _mean_pool(hf, batch_ref)
    combined = jnp.concatenate([pa, pf], axis=1)
    z = jax.nn.relu(jnp.dot(combined, wc1_ref[...],
                            preferred_element_type=jnp.float32)
                    + bc1_ref[...][None, :])
    out_ref[...] = (jnp.dot(z, wc2_ref[...], preferred_element_type=jnp.float32)
                    + bc2_ref[...][None, :])


_tc_final = pl.pallas_call(
    _tc_final_body,
    out_shape=jax.ShapeDtypeStruct((G, OUT), jnp.float32),
)


def _cast_edges(edge_index, edge_attr):
    src = edge_index[0].astype(jnp.int32)
    dst = edge_index[1].astype(jnp.int32)
    w = edge_attr[:, 0].astype(jnp.float32)
    pad = EP - E
    pidx = (jnp.arange(pad, dtype=jnp.int32) * 37) % N
    src = jnp.concatenate([src, pidx])
    dst = jnp.concatenate([dst, pidx])
    w = jnp.concatenate([w, jnp.zeros((pad,), jnp.float32)])
    return src, dst, w


def kernel(x, func_edge_index, func_edge_attr, anat_edge_index, anat_edge_attr,
           batch, W_s0, b_s0, g_s0, be_s0, W_s1, b_s1, g_s1, be_s1,
           W_a, b_a, g_a, be_a, W_f, b_f, g_f, be_f, Wc1, bc1, Wc2, bc2):
    fsrc, fdst, fwt = _cast_edges(func_edge_index, func_edge_attr)
    asrc, adst, awt = _cast_edges(anat_edge_index, anat_edge_attr)
    fs = fsrc.reshape(NW, NCHUNK, CH)
    fd = fdst.reshape(NW, NCHUNK, CH)
    fw = fwt.reshape(NW, NCHUNK, CH)
    src2 = jnp.stack([fsrc, asrc]).reshape(NC, NS, C2, CH)
    dst2 = jnp.stack([fdst, adst]).reshape(NC, NS, C2, CH)
    w2 = jnp.stack([fwt, awt]).reshape(NC, NS, C2, CH)
    batch2d = batch.astype(jnp.int32)[:, None]

    degp = _deg_kernel()(dst2, w2)
    hw0, g0, dinvf, dinva = _tc_prep(x, W_s0, degp)

    spmm = _spmm_kernel()
    aggp0 = spmm(g0, fs, fd, fw)
    hw1, g1 = _tc_mid0(aggp0, hw0, dinvf, b_s0, g_s0, be_s0, W_s1)

    aggp1 = spmm(g1, fs, fd, fw)
    hwa, hwf, g2 = _tc_mid1(aggp1, hw1, dinvf, dinva, b_s1, g_s1, be_s1,
                            W_a, W_f)

    agg2 = _spmm2_kernel()(g2, src2, dst2, w2)

    return _tc_final(agg2, hwa, hwf, dinva, dinvf, b_a, g_a, be_a,
                     b_f, g_f, be_f, batch2d, Wc1, bc1, Wc2, bc2)


# speedup vs baseline: 1.1000x; 1.0311x over previous
import functools

import jax
import jax.numpy as jnp
from jax import lax
from jax.experimental import pallas as pl
from jax.experimental.pallas import tpu as pltpu
from jax.experimental.pallas import tpu_sc as plsc

N = 10000
E = 320000
D = 128
H = 64
G = 16
OUT = 2

NC = 2
NS = 16
LN = 16
NW = NC * NS

CH = 128
NCHUNK = 80
EP = NW * NCHUNK * CH
C2 = 160
NP = 10240
RPT = NP // NS

@functools.cache
def _sc_mesh():
    return plsc.VectorSubcoreMesh(core_axis_name="c", subcore_axis_name="s",
                                  num_cores=NC, num_subcores=NS)


def _zero_rows(rows):
    z16 = jnp.zeros((LN,), jnp.float32)

    def body(i, carry):
        r = i // (H // LN)
        q = i % (H // LN)
        rows[r, pl.ds(q * LN, LN)] = z16
        return carry

    lax.fori_loop(0, CH * (H // LN), body, 0, unroll=8)


def _deg_body(idx_hbm, w_hbm, out_hbm, acc, idx_v, w_v, zrow):
    cid = lax.axis_index("c")
    sid = lax.axis_index("s")

    z16 = jnp.zeros((LN,), jnp.float32)

    def zb(i, c):
        zrow[pl.ds(i * LN, LN)] = z16
        return c

    lax.fori_loop(0, RPT // LN, zb, 0, unroll=8)
    pltpu.sync_copy(zrow, acc.at[pl.ds(sid * RPT, RPT)])
    plsc.subcore_barrier()

    pltpu.sync_copy(idx_hbm.at[cid, sid], idx_v)
    pltpu.sync_copy(w_hbm.at[cid, sid], w_v)

    def body(j, c):
        pltpu.sync_copy(w_v.at[j], acc.at[idx_v.at[j]], add=True)
        return c

    lax.fori_loop(0, C2, body, 0)
    plsc.subcore_barrier()
    pltpu.sync_copy(acc.at[pl.ds(sid * RPT, RPT)],
                    out_hbm.at[cid, pl.ds(sid * RPT, RPT)])


@functools.cache
def _deg_kernel():
    return pl.kernel(
        _deg_body,
        out_type=jax.ShapeDtypeStruct((NC, NP), jnp.float32),
        mesh=_sc_mesh(),
        compiler_params=pltpu.CompilerParams(use_tc_tiling_on_sc=False),
        scratch_types=[
            pltpu.VMEM_SHARED((NP,), jnp.float32),
            pltpu.VMEM((C2, CH), jnp.int32),
            pltpu.VMEM((C2, CH), jnp.float32),
            pltpu.VMEM((RPT,), jnp.float32),
        ],
    )


def _scale_rows(rows, rowsf, w_v, j):
    for gi in range(CH // LN):
        wv = w_v[j, pl.ds(gi * LN, LN)]
        for ee in range(LN):
            wb = jnp.full((LN,), wv[ee], jnp.float32)
            e = gi * LN + ee
            for q in range(H // LN):
                rowsf[e, pl.ds(q * LN, LN)] = rows[e, pl.ds(q * LN, LN)] * wb


def _spmm_loop(g_hbm, acc, sidx_v, didx_v, w_v, rows0, rows1, rowsf0, rowsf1,
               sem0, sem1, ssem0, ssem1, nchunk):
    pltpu.async_copy(g_hbm.at[sidx_v.at[0]], rows0, sem0)
    pltpu.async_copy(g_hbm.at[sidx_v.at[1]], rows1, sem1)

    def body(jj, c):
        j0 = 2 * jj
        pltpu.make_async_copy(g_hbm.at[sidx_v.at[j0]], rows0, sem0).wait()

        @pl.when(j0 >= 2)
        def _():
            pltpu.make_async_copy(rowsf0, acc.at[didx_v.at[j0 - 2]],
                                  ssem0).wait()

        _scale_rows(rows0, rowsf0, w_v, j0)
        pltpu.async_copy(rowsf0, acc.at[didx_v.at[j0]], ssem0, add=True)

        @pl.when(j0 + 2 < nchunk)
        def _():
            pltpu.async_copy(g_hbm.at[sidx_v.at[j0 + 2]], rows0, sem0)

        pltpu.make_async_copy(g_hbm.at[sidx_v.at[j0 + 1]], rows1, sem1).wait()

        @pl.when(j0 >= 2)
        def _():
            pltpu.make_async_copy(rowsf1, acc.at[didx_v.at[j0 - 1]],
                                  ssem1).wait()

        _scale_rows(rows1, rowsf1, w_v, j0 + 1)
        pltpu.async_copy(rowsf1, acc.at[didx_v.at[j0 + 1]], ssem1, add=True)

        @pl.when(j0 + 3 < nchunk)
        def _():
            pltpu.async_copy(g_hbm.at[sidx_v.at[j0 + 3]], rows1, sem1)

        return c

    lax.fori_loop(0, nchunk // 2, body, 0)

    pltpu.make_async_copy(rowsf0, acc.at[didx_v.at[nchunk - 2]], ssem0).wait()
    pltpu.make_async_copy(rowsf1, acc.at[didx_v.at[nchunk - 1]], ssem1).wait()


def _spmm_body(g_hbm, sidx_hbm, didx_hbm, w_hbm, out_hbm,
               acc, sidx_v, didx_v, w_v, rows0, rows1, rowsf0, rowsf1,
               sem0, sem1, ssem0, ssem1):
    cid = lax.axis_index("c")
    sid = lax.axis_index("s")
    wid = sid * NC + cid

    pltpu.sync_copy(sidx_hbm.at[wid], sidx_v)
    pltpu.sync_copy(didx_hbm.at[wid], didx_v)
    pltpu.sync_copy(w_hbm.at[wid], w_v)

    _zero_rows(rowsf0)
    for k in range(RPT // CH):
        pltpu.sync_copy(rowsf0, acc.at[pl.ds(sid * RPT + k * CH, CH)])
    plsc.subcore_barrier()

    _spmm_loop(g_hbm, acc, sidx_v, didx_v, w_v, rows0, rows1, rowsf0, rowsf1,
               sem0, sem1, ssem0, ssem1, NCHUNK)

    plsc.subcore_barrier()
    pltpu.sync_copy(acc.at[pl.ds(sid * RPT, RPT)],
                    out_hbm.at[cid, pl.ds(sid * RPT, RPT)])


@functools.cache
def _spmm_kernel():
    return pl.kernel(
        _spmm_body,
        out_type=jax.ShapeDtypeStruct((NC, NP, H), jnp.float32),
        mesh=_sc_mesh(),
        compiler_params=pltpu.CompilerParams(use_tc_tiling_on_sc=False),
        scratch_types=[
            pltpu.VMEM_SHARED((NP, H), jnp.float32),
            pltpu.VMEM((NCHUNK, CH), jnp.int32),
            pltpu.VMEM((NCHUNK, CH), jnp.int32),
            pltpu.VMEM((NCHUNK, CH), jnp.float32),
            pltpu.VMEM((CH, H), jnp.float32),
            pltpu.VMEM((CH, H), jnp.float32),
            pltpu.VMEM((CH, H), jnp.float32),
            pltpu.VMEM((CH, H), jnp.float32),
            pltpu.SemaphoreType.DMA,
            pltpu.SemaphoreType.DMA,
            pltpu.SemaphoreType.DMA,
            pltpu.SemaphoreType.DMA,
        ],
    )


def _spmm2_body(g2_hbm, sidx_hbm, didx_hbm, w_hbm, out_hbm,
                acc, sidx_v, didx_v, w_v, rows0, rows1, rowsf0, rowsf1,
                sem0, sem1, ssem0, ssem1):
    cid = lax.axis_index("c")
    sid = lax.axis_index("s")
    g_hbm = g2_hbm.at[cid]
    HC = C2 // 2

    _zero_rows(rowsf0)
    for k in range(RPT // CH):
        pltpu.sync_copy(rowsf0, acc.at[pl.ds(sid * RPT + k * CH, CH)])
    plsc.subcore_barrier()

    for half in range(2):
        base = half * HC
        pltpu.sync_copy(sidx_hbm.at[cid, sid, pl.ds(base, HC)], sidx_v)
        pltpu.sync_copy(didx_hbm.at[cid, sid, pl.ds(base, HC)], didx_v)
        pltpu.sync_copy(w_hbm.at[cid, sid, pl.ds(base, HC)], w_v)
        _spmm_loop(g_hbm, acc, sidx_v, didx_v, w_v, rows0, rows1,
                   rowsf0, rowsf1, sem0, sem1, ssem0, ssem1, HC)

    plsc.subcore_barrier()
    pltpu.sync_copy(acc.at[pl.ds(sid * RPT, RPT)],
                    out_hbm.at[cid, pl.ds(sid * RPT, RPT)])


@functools.cache
def _spmm2_kernel():
    return pl.kernel(
        _spmm2_body,
        out_type=jax.ShapeDtypeStruct((NC, NP, H), jnp.float32),
        mesh=_sc_mesh(),
        compiler_params=pltpu.CompilerParams(use_tc_tiling_on_sc=False),
        scratch_types=[
            pltpu.VMEM_SHARED((NP, H), jnp.float32),
            pltpu.VMEM((C2 // 2, CH), jnp.int32),
            pltpu.VMEM((C2 // 2, CH), jnp.int32),
            pltpu.VMEM((C2 // 2, CH), jnp.float32),
            pltpu.VMEM((CH, H), jnp.float32),
            pltpu.VMEM((CH, H), jnp.float32),
            pltpu.VMEM((CH, H), jnp.float32),
            pltpu.VMEM((CH, H), jnp.float32),
            pltpu.SemaphoreType.DMA,
            pltpu.SemaphoreType.DMA,
            pltpu.SemaphoreType.DMA,
            pltpu.SemaphoreType.DMA,
        ],
    )


def _bn_cols(y, g, be):
    m = jnp.mean(y, axis=0, keepdims=True)
    v = jnp.mean(y * y, axis=0, keepdims=True) - m * m
    return (y - m) * lax.rsqrt(v + 1e-5) * g[None, :] + be[None, :]


def _tc_prep_body(x_ref, w_ref, degp_ref, hw_ref, g_ref, dinvf_ref, dinva_ref):
    degf = degp_ref[0, :N] + 1.0
    dega = degp_ref[1, :N] + 1.0
    dinvf = jnp.where(degf > 0, lax.rsqrt(degf), 0.0)[:, None]
    dinva = jnp.where(dega > 0, lax.rsqrt(dega), 0.0)[:, None]
    hw = jnp.dot(x_ref[...], w_ref[...], preferred_element_type=jnp.float32)
    hw_ref[...] = hw
    g_ref[...] = dinvf * hw
    dinvf_ref[...] = dinvf
    dinva_ref[...] = dinva


_tc_prep = pl.pallas_call(
    _tc_prep_body,
    out_shape=[
        jax.ShapeDtypeStruct((N, H), jnp.float32),
        jax.ShapeDtypeStruct((N, H), jnp.float32),
        jax.ShapeDtypeStruct((N, 1), jnp.float32),
        jax.ShapeDtypeStruct((N, 1), jnp.float32),
    ],
)


def _post(aggp_ref, hw_ref, dinv_ref, b_ref, g_ref, be_ref):
    dinv = dinv_ref[...]
    agg = aggp_ref[0, :N, :] + aggp_ref[1, :N, :]
    y = dinv * agg + (dinv * dinv) * hw_ref[...] + b_ref[...][None, :]
    return _bn_cols(y, g_ref[...], be_ref[...])


def _tc_mid0_body(aggp_ref, hw_ref, dinvf_ref, b_ref, g_ref, be_ref, w1_ref,
                  hw1_ref, g1_ref):
    h0 = jax.nn.relu(_post(aggp_ref, hw_ref, dinvf_ref, b_ref, g_ref, be_ref))
    hw1 = jnp.dot(h0, w1_ref[...], preferred_element_type=jnp.float32)
    hw1_ref[...] = hw1
    g1_ref[...] = dinvf_ref[...] * hw1


_tc_mid0 = pl.pallas_call(
    _tc_mid0_body,
    out_shape=[
        jax.ShapeDtypeStruct((N, H), jnp.float32),
        jax.ShapeDtypeStruct((N, H), jnp.float32),
    ],
)


def _tc_mid1_body(aggp_ref, hw_ref, dinvf_ref, dinva_ref, b_ref, g_ref, be_ref,
                  wa_ref, wf_ref, hwa_ref, hwf_ref, g2_ref):
    h1 = _post(aggp_ref, hw_ref, dinvf_ref, b_ref, g_ref, be_ref)
    hwa = jnp.dot(h1, wa_ref[...], preferred_element_type=jnp.float32)
    hwa_ref[...] = hwa
    hwf = jnp.dot(h1, wf_ref[...], preferred_element_type=jnp.float32)
    hwf_ref[...] = hwf
    g2_ref[0] = dinvf_ref[...] * hwf
    g2_ref[1] = dinva_ref[...] * hwa


_tc_mid1 = pl.pallas_call(
    _tc_mid1_body,
    out_shape=[
        jax.ShapeDtypeStruct((N, H), jnp.float32),
        jax.ShapeDtypeStruct((N, H), jnp.float32),
        jax.ShapeDtypeStruct((NC, N, H), jnp.float32),
    ],
)


def _post1(agg_ref, hw_ref, dinv_ref, b_ref, g_ref, be_ref):
    dinv = dinv_ref[...]
    y = (dinv * agg_ref[:N, :] + (dinv * dinv) * hw_ref[...]
         + b_ref[...][None, :])
    return _bn_cols(y, g_ref[...], be_ref[...])


def _mean_pool(h, batch_ref):
    oh = (batch_ref[...] == lax.broadcasted_iota(jnp.int32, (1, G), 1)
          ).astype(jnp.float32)
    cnt = jnp.maximum(jnp.sum(oh, axis=0), 1.0)[:, None]
    dn = (((0,), (0,)), ((), ()))
    return lax.dot_general(oh, h, dn, preferred_element_type=jnp.float32) / cnt


def _tc_final_body(agg2_ref, hwa_ref, hwf_ref, dinva_ref, dinvf_ref,
                   ba_ref, ga_ref, bea_ref, bf_ref, gf_ref, bef_ref,
                   batch_ref, wc1_ref, bc1_ref, wc2_ref, bc2_ref, out_ref):
    ha = _post1(agg2_ref.at[1], hwa_ref, dinva_ref, ba_ref, ga_ref, bea_ref)
    pa = _mean_pool(ha, batch_ref)
    hf = _post1(agg2_ref.at[0], hwf_ref, dinvf_ref, bf_ref, gf_ref, bef_ref)
    pf = _mean_pool(hf, batch_ref)
    combined = jnp.concatenate([pa, pf], axis=1)
    z = jax.nn.relu(jnp.dot(combined, wc1_ref[...],
                            preferred_element_type=jnp.float32)
                    + bc1_ref[...][None, :])
    out_ref[...] = (jnp.dot(z, wc2_ref[...], preferred_element_type=jnp.float32)
                    + bc2_ref[...][None, :])


_tc_final = pl.pallas_call(
    _tc_final_body,
    out_shape=jax.ShapeDtypeStruct((G, OUT), jnp.float32),
)


def _cast_edges(edge_index, edge_attr):
    src = edge_index[0].astype(jnp.int32)
    dst = edge_index[1].astype(jnp.int32)
    w = edge_attr[:, 0].astype(jnp.float32)
    pad = EP - E
    pidx = (jnp.arange(pad, dtype=jnp.int32) * 37) % N
    src = jnp.concatenate([src, pidx])
    dst = jnp.concatenate([dst, pidx])
    w = jnp.concatenate([w, jnp.zeros((pad,), jnp.float32)])
    return src, dst, w


def kernel(x, func_edge_index, func_edge_attr, anat_edge_index, anat_edge_attr,
           batch, W_s0, b_s0, g_s0, be_s0, W_s1, b_s1, g_s1, be_s1,
           W_a, b_a, g_a, be_a, W_f, b_f, g_f, be_f, Wc1, bc1, Wc2, bc2):
    fsrc, fdst, fwt = _cast_edges(func_edge_index, func_edge_attr)
    asrc, adst, awt = _cast_edges(anat_edge_index, anat_edge_attr)
    fs = fsrc.reshape(NW, NCHUNK, CH)
    fd = fdst.reshape(NW, NCHUNK, CH)
    fw = fwt.reshape(NW, NCHUNK, CH)
    src2 = jnp.stack([fsrc, asrc]).reshape(NC, NS, C2, CH)
    dst2 = jnp.stack([fdst, adst]).reshape(NC, NS, C2, CH)
    w2 = jnp.stack([fwt, awt]).reshape(NC, NS, C2, CH)
    batch2d = batch.astype(jnp.int32)[:, None]

    degp = _deg_kernel()(dst2, w2)
    hw0, g0, dinvf, dinva = _tc_prep(x, W_s0, degp)

    spmm = _spmm_kernel()
    aggp0 = spmm(g0, fs, fd, fw)
    hw1, g1 = _tc_mid0(aggp0, hw0, dinvf, b_s0, g_s0, be_s0, W_s1)

    aggp1 = spmm(g1, fs, fd, fw)
    hwa, hwf, g2 = _tc_mid1(aggp1, hw1, dinvf, dinva, b_s1, g_s1, be_s1,
                            W_a, W_f)

    agg2 = _spmm2_kernel()(g2, src2, dst2, w2)

    return _tc_final(agg2, hwa, hwf, dinva, dinvf, b_a, g_a, be_a,
                     b_f, g_f, be_f, batch2d, Wc1, bc1, Wc2, bc2)


# speedup vs baseline: 1.1270x; 1.0245x over previous
import functools

import jax
import jax.numpy as jnp
from jax import lax
from jax.experimental import pallas as pl
from jax.experimental.pallas import tpu as pltpu
from jax.experimental.pallas import tpu_sc as plsc

N = 10000
E = 320000
D = 128
H = 64
G = 16
OUT = 2

NC = 2
NS = 16
LN = 16
NW = NC * NS

CH = 128
NCHUNK = 80
EP = NW * NCHUNK * CH
C2 = 160
NP = 10240
RPT = NP // NS

@functools.cache
def _sc_mesh():
    return plsc.VectorSubcoreMesh(core_axis_name="c", subcore_axis_name="s",
                                  num_cores=NC, num_subcores=NS)


def _zero_rows(rows):
    z16 = jnp.zeros((LN,), jnp.float32)

    def body(i, carry):
        r = i // (H // LN)
        q = i % (H // LN)
        rows[r, pl.ds(q * LN, LN)] = z16
        return carry

    lax.fori_loop(0, CH * (H // LN), body, 0, unroll=8)


def _deg_body(idx_hbm, w_hbm, out_hbm, acc, idx_v, w_v, zrow, dsem):
    cid = lax.axis_index("c")
    sid = lax.axis_index("s")

    z16 = jnp.zeros((LN,), jnp.float32)

    def zb(i, c):
        zrow[pl.ds(i * LN, LN)] = z16
        return c

    lax.fori_loop(0, RPT // LN, zb, 0, unroll=8)
    pltpu.sync_copy(zrow, acc.at[pl.ds(sid * RPT, RPT)])
    plsc.subcore_barrier()

    pltpu.sync_copy(idx_hbm.at[cid, sid], idx_v)
    pltpu.sync_copy(w_hbm.at[cid, sid], w_v)

    def body(i, c):
        for b in range(8):
            pltpu.async_copy(w_v.at[8 * i + b], acc.at[idx_v.at[8 * i + b]],
                             dsem, add=True)
        for b in range(8):
            pltpu.make_async_copy(w_v.at[8 * i + b],
                                  acc.at[idx_v.at[8 * i + b]], dsem).wait()
        return c

    lax.fori_loop(0, C2 // 8, body, 0)
    plsc.subcore_barrier()
    pltpu.sync_copy(acc.at[pl.ds(sid * RPT, RPT)],
                    out_hbm.at[cid, pl.ds(sid * RPT, RPT)])


@functools.cache
def _deg_kernel():
    return pl.kernel(
        _deg_body,
        out_type=jax.ShapeDtypeStruct((NC, NP), jnp.float32),
        mesh=_sc_mesh(),
        compiler_params=pltpu.CompilerParams(use_tc_tiling_on_sc=False),
        scratch_types=[
            pltpu.VMEM_SHARED((NP,), jnp.float32),
            pltpu.VMEM((C2, CH), jnp.int32),
            pltpu.VMEM((C2, CH), jnp.float32),
            pltpu.VMEM((RPT,), jnp.float32),
            pltpu.SemaphoreType.DMA,
        ],
    )


def _scale_rows(rows, rowsf, w_v, j):
    for gi in range(CH // LN):
        wv = w_v[j, pl.ds(gi * LN, LN)]
        for ee in range(LN):
            wb = jnp.full((LN,), wv[ee], jnp.float32)
            e = gi * LN + ee
            for q in range(H // LN):
                rowsf[e, pl.ds(q * LN, LN)] = rows[e, pl.ds(q * LN, LN)] * wb


def _spmm_loop(g_hbm, acc, sidx_v, didx_v, w_v, rows0, rows1, rowsf0, rowsf1,
               sem0, sem1, ssem0, ssem1, nchunk, prime=True):
    if prime:
        pltpu.async_copy(g_hbm.at[sidx_v.at[0]], rows0, sem0)
        pltpu.async_copy(g_hbm.at[sidx_v.at[1]], rows1, sem1)

    def body(jj, c):
        j0 = 2 * jj
        pltpu.make_async_copy(g_hbm.at[sidx_v.at[j0]], rows0, sem0).wait()

        @pl.when(j0 >= 2)
        def _():
            pltpu.make_async_copy(rowsf0, acc.at[didx_v.at[j0 - 2]],
                                  ssem0).wait()

        _scale_rows(rows0, rowsf0, w_v, j0)
        pltpu.async_copy(rowsf0, acc.at[didx_v.at[j0]], ssem0, add=True)

        @pl.when(j0 + 2 < nchunk)
        def _():
            pltpu.async_copy(g_hbm.at[sidx_v.at[j0 + 2]], rows0, sem0)

        pltpu.make_async_copy(g_hbm.at[sidx_v.at[j0 + 1]], rows1, sem1).wait()

        @pl.when(j0 >= 2)
        def _():
            pltpu.make_async_copy(rowsf1, acc.at[didx_v.at[j0 - 1]],
                                  ssem1).wait()

        _scale_rows(rows1, rowsf1, w_v, j0 + 1)
        pltpu.async_copy(rowsf1, acc.at[didx_v.at[j0 + 1]], ssem1, add=True)

        @pl.when(j0 + 3 < nchunk)
        def _():
            pltpu.async_copy(g_hbm.at[sidx_v.at[j0 + 3]], rows1, sem1)

        return c

    lax.fori_loop(0, nchunk // 2, body, 0)

    pltpu.make_async_copy(rowsf0, acc.at[didx_v.at[nchunk - 2]], ssem0).wait()
    pltpu.make_async_copy(rowsf1, acc.at[didx_v.at[nchunk - 1]], ssem1).wait()


def _spmm_body(g_hbm, sidx_hbm, didx_hbm, w_hbm, out_hbm,
               acc, sidx_v, didx_v, w_v, rows0, rows1, rowsf0, rowsf1,
               sem0, sem1, ssem0, ssem1):
    cid = lax.axis_index("c")
    sid = lax.axis_index("s")
    wid = sid * NC + cid

    pltpu.sync_copy(sidx_hbm.at[wid], sidx_v)
    pltpu.sync_copy(didx_hbm.at[wid], didx_v)
    pltpu.sync_copy(w_hbm.at[wid], w_v)

    pltpu.async_copy(g_hbm.at[sidx_v.at[0]], rows0, sem0)
    pltpu.async_copy(g_hbm.at[sidx_v.at[1]], rows1, sem1)

    _zero_rows(rowsf0)
    for k in range(RPT // CH):
        pltpu.sync_copy(rowsf0, acc.at[pl.ds(sid * RPT + k * CH, CH)])
    plsc.subcore_barrier()

    _spmm_loop(g_hbm, acc, sidx_v, didx_v, w_v, rows0, rows1, rowsf0, rowsf1,
               sem0, sem1, ssem0, ssem1, NCHUNK, prime=False)

    plsc.subcore_barrier()
    pltpu.sync_copy(acc.at[pl.ds(sid * RPT, RPT)],
                    out_hbm.at[cid, pl.ds(sid * RPT, RPT)])


@functools.cache
def _spmm_kernel():
    return pl.kernel(
        _spmm_body,
        out_type=jax.ShapeDtypeStruct((NC, NP, H), jnp.float32),
        mesh=_sc_mesh(),
        compiler_params=pltpu.CompilerParams(use_tc_tiling_on_sc=False),
        scratch_types=[
            pltpu.VMEM_SHARED((NP, H), jnp.float32),
            pltpu.VMEM((NCHUNK, CH), jnp.int32),
            pltpu.VMEM((NCHUNK, CH), jnp.int32),
            pltpu.VMEM((NCHUNK, CH), jnp.float32),
            pltpu.VMEM((CH, H), jnp.float32),
            pltpu.VMEM((CH, H), jnp.float32),
            pltpu.VMEM((CH, H), jnp.float32),
            pltpu.VMEM((CH, H), jnp.float32),
            pltpu.SemaphoreType.DMA,
            pltpu.SemaphoreType.DMA,
            pltpu.SemaphoreType.DMA,
            pltpu.SemaphoreType.DMA,
        ],
    )


def _spmm2_body(g2_hbm, sidx_hbm, didx_hbm, w_hbm, out_hbm,
                acc, sidx_v, didx_v, w_v, rows0, rows1, rowsf0, rowsf1,
                sem0, sem1, ssem0, ssem1):
    cid = lax.axis_index("c")
    sid = lax.axis_index("s")
    g_hbm = g2_hbm.at[cid]
    HC = C2 // 2

    pltpu.sync_copy(sidx_hbm.at[cid, sid, pl.ds(0, HC)], sidx_v)
    pltpu.sync_copy(didx_hbm.at[cid, sid, pl.ds(0, HC)], didx_v)
    pltpu.sync_copy(w_hbm.at[cid, sid, pl.ds(0, HC)], w_v)
    pltpu.async_copy(g_hbm.at[sidx_v.at[0]], rows0, sem0)
    pltpu.async_copy(g_hbm.at[sidx_v.at[1]], rows1, sem1)

    _zero_rows(rowsf0)
    for k in range(RPT // CH):
        pltpu.sync_copy(rowsf0, acc.at[pl.ds(sid * RPT + k * CH, CH)])
    plsc.subcore_barrier()

    _spmm_loop(g_hbm, acc, sidx_v, didx_v, w_v, rows0, rows1,
               rowsf0, rowsf1, sem0, sem1, ssem0, ssem1, HC, prime=False)
    pltpu.sync_copy(sidx_hbm.at[cid, sid, pl.ds(HC, HC)], sidx_v)
    pltpu.sync_copy(didx_hbm.at[cid, sid, pl.ds(HC, HC)], didx_v)
    pltpu.sync_copy(w_hbm.at[cid, sid, pl.ds(HC, HC)], w_v)
    _spmm_loop(g_hbm, acc, sidx_v, didx_v, w_v, rows0, rows1,
               rowsf0, rowsf1, sem0, sem1, ssem0, ssem1, HC)

    plsc.subcore_barrier()
    pltpu.sync_copy(acc.at[pl.ds(sid * RPT, RPT)],
                    out_hbm.at[cid, pl.ds(sid * RPT, RPT)])


@functools.cache
def _spmm2_kernel():
    return pl.kernel(
        _spmm2_body,
        out_type=jax.ShapeDtypeStruct((NC, NP, H), jnp.float32),
        mesh=_sc_mesh(),
        compiler_params=pltpu.CompilerParams(use_tc_tiling_on_sc=False),
        scratch_types=[
            pltpu.VMEM_SHARED((NP, H), jnp.float32),
            pltpu.VMEM((C2 // 2, CH), jnp.int32),
            pltpu.VMEM((C2 // 2, CH), jnp.int32),
            pltpu.VMEM((C2 // 2, CH), jnp.float32),
            pltpu.VMEM((CH, H), jnp.float32),
            pltpu.VMEM((CH, H), jnp.float32),
            pltpu.VMEM((CH, H), jnp.float32),
            pltpu.VMEM((CH, H), jnp.float32),
            pltpu.SemaphoreType.DMA,
            pltpu.SemaphoreType.DMA,
            pltpu.SemaphoreType.DMA,
            pltpu.SemaphoreType.DMA,
        ],
    )


def _bn_cols(y, g, be):
    m = jnp.mean(y, axis=0, keepdims=True)
    v = jnp.mean(y * y, axis=0, keepdims=True) - m * m
    return (y - m) * lax.rsqrt(v + 1e-5) * g[None, :] + be[None, :]


def _tc_prep_body(x_ref, w_ref, degp_ref, hw_ref, g_ref, dinvf_ref, dinva_ref):
    degf = degp_ref[0, :N] + 1.0
    dega = degp_ref[1, :N] + 1.0
    dinvf = jnp.where(degf > 0, lax.rsqrt(degf), 0.0)[:, None]
    dinva = jnp.where(dega > 0, lax.rsqrt(dega), 0.0)[:, None]
    hw = jnp.dot(x_ref[...], w_ref[...], preferred_element_type=jnp.float32)
    hw_ref[...] = hw
    g_ref[...] = dinvf * hw
    dinvf_ref[...] = dinvf
    dinva_ref[...] = dinva


_tc_prep = pl.pallas_call(
    _tc_prep_body,
    out_shape=[
        jax.ShapeDtypeStruct((N, H), jnp.float32),
        jax.ShapeDtypeStruct((N, H), jnp.float32),
        jax.ShapeDtypeStruct((N, 1), jnp.float32),
        jax.ShapeDtypeStruct((N, 1), jnp.float32),
    ],
)


def _post(aggp_ref, hw_ref, dinv_ref, b_ref, g_ref, be_ref):
    dinv = dinv_ref[...]
    agg = aggp_ref[0, :N, :] + aggp_ref[1, :N, :]
    y = dinv * agg + (dinv * dinv) * hw_ref[...] + b_ref[...][None, :]
    return _bn_cols(y, g_ref[...], be_ref[...])


def _tc_mid0_body(aggp_ref, hw_ref, dinvf_ref, b_ref, g_ref, be_ref, w1_ref,
                  hw1_ref, g1_ref):
    h0 = jax.nn.relu(_post(aggp_ref, hw_ref, dinvf_ref, b_ref, g_ref, be_ref))
    hw1 = jnp.dot(h0, w1_ref[...], preferred_element_type=jnp.float32)
    hw1_ref[...] = hw1
    g1_ref[...] = dinvf_ref[...] * hw1


_tc_mid0 = pl.pallas_call(
    _tc_mid0_body,
    out_shape=[
        jax.ShapeDtypeStruct((N, H), jnp.float32),
        jax.ShapeDtypeStruct((N, H), jnp.float32),
    ],
)


def _tc_mid1_body(aggp_ref, hw_ref, dinvf_ref, dinva_ref, b_ref, g_ref, be_ref,
                  wa_ref, wf_ref, hwa_ref, hwf_ref, g2_ref):
    h1 = _post(aggp_ref, hw_ref, dinvf_ref, b_ref, g_ref, be_ref)
    hwa = jnp.dot(h1, wa_ref[...], preferred_element_type=jnp.float32)
    hwa_ref[...] = hwa
    hwf = jnp.dot(h1, wf_ref[...], preferred_element_type=jnp.float32)
    hwf_ref[...] = hwf
    g2_ref[0] = dinvf_ref[...] * hwf
    g2_ref[1] = dinva_ref[...] * hwa


_tc_mid1 = pl.pallas_call(
    _tc_mid1_body,
    out_shape=[
        jax.ShapeDtypeStruct((N, H), jnp.float32),
        jax.ShapeDtypeStruct((N, H), jnp.float32),
        jax.ShapeDtypeStruct((NC, N, H), jnp.float32),
    ],
)


def _post1(agg_ref, hw_ref, dinv_ref, b_ref, g_ref, be_ref):
    dinv = dinv_ref[...]
    y = (dinv * agg_ref[:N, :] + (dinv * dinv) * hw_ref[...]
         + b_ref[...][None, :])
    return _bn_cols(y, g_ref[...], be_ref[...])


def _mean_pool(h, batch_ref):
    oh = (batch_ref[...] == lax.broadcasted_iota(jnp.int32, (1, G), 1)
          ).astype(jnp.float32)
    cnt = jnp.maximum(jnp.sum(oh, axis=0), 1.0)[:, None]
    dn = (((0,), (0,)), ((), ()))
    return lax.dot_general(oh, h, dn, preferred_element_type=jnp.float32) / cnt


def _tc_final_body(agg2_ref, hwa_ref, hwf_ref, dinva_ref, dinvf_ref,
                   ba_ref, ga_ref, bea_ref, bf_ref, gf_ref, bef_ref,
                   batch_ref, wc1_ref, bc1_ref, wc2_ref, bc2_ref, out_ref):
    ha = _post1(agg2_ref.at[1], hwa_ref, dinva_ref, ba_ref, ga_ref, bea_ref)
    pa = _mean_pool(ha, batch_ref)
    hf = _post1(agg2_ref.at[0], hwf_ref, dinvf_ref, bf_ref, gf_ref, bef_ref)
    pf = _mean_pool(hf, batch_ref)
    combined = jnp.concatenate([pa, pf], axis=1)
    z = jax.nn.relu(jnp.dot(combined, wc1_ref[...],
                            preferred_element_type=jnp.float32)
                    + bc1_ref[...][None, :])
    out_ref[...] = (jnp.dot(z, wc2_ref[...], preferred_element_type=jnp.float32)
                    + bc2_ref[...][None, :])


_tc_final = pl.pallas_call(
    _tc_final_body,
    out_shape=jax.ShapeDtypeStruct((G, OUT), jnp.float32),
)


def _cast_edges(edge_index, edge_attr):
    src = edge_index[0].astype(jnp.int32)
    dst = edge_index[1].astype(jnp.int32)
    w = edge_attr[:, 0].astype(jnp.float32)
    pad = EP - E
    pidx = (jnp.arange(pad, dtype=jnp.int32) * 37) % N
    src = jnp.concatenate([src, pidx])
    dst = jnp.concatenate([dst, pidx])
    w = jnp.concatenate([w, jnp.zeros((pad,), jnp.float32)])
    return src, dst, w


def kernel(x, func_edge_index, func_edge_attr, anat_edge_index, anat_edge_attr,
           batch, W_s0, b_s0, g_s0, be_s0, W_s1, b_s1, g_s1, be_s1,
           W_a, b_a, g_a, be_a, W_f, b_f, g_f, be_f, Wc1, bc1, Wc2, bc2):
    fsrc, fdst, fwt = _cast_edges(func_edge_index, func_edge_attr)
    asrc, adst, awt = _cast_edges(anat_edge_index, anat_edge_attr)
    fs = fsrc.reshape(NW, NCHUNK, CH)
    fd = fdst.reshape(NW, NCHUNK, CH)
    fw = fwt.reshape(NW, NCHUNK, CH)
    src2 = jnp.stack([fsrc, asrc]).reshape(NC, NS, C2, CH)
    dst2 = jnp.stack([fdst, adst]).reshape(NC, NS, C2, CH)
    w2 = jnp.stack([fwt, awt]).reshape(NC, NS, C2, CH)
    batch2d = batch.astype(jnp.int32)[:, None]

    degp = _deg_kernel()(dst2, w2)
    hw0, g0, dinvf, dinva = _tc_prep(x, W_s0, degp)

    spmm = _spmm_kernel()
    aggp0 = spmm(g0, fs, fd, fw)
    hw1, g1 = _tc_mid0(aggp0, hw0, dinvf, b_s0, g_s0, be_s0, W_s1)

    aggp1 = spmm(g1, fs, fd, fw)
    hwa, hwf, g2 = _tc_mid1(aggp1, hw1, dinvf, dinva, b_s1, g_s1, be_s1,
                            W_a, W_f)

    agg2 = _spmm2_kernel()(g2, src2, dst2, w2)

    return _tc_final(agg2, hwa, hwf, dinva, dinvf, b_a, g_a, be_a,
                     b_f, g_f, be_f, batch2d, Wc1, bc1, Wc2, bc2)
